# Initial kernel scaffold; baseline (speedup 1.0000x reference)
import functools

import jax
import jax.numpy as jnp
from jax import lax
from jax.experimental import pallas as pl
from jax.experimental.pallas import tpu as pltpu
from jax.experimental.pallas import tpu_sc as plsc

NC = 2
NS = 16
NW = NC * NS
CHUNK = 128


def _make_lookup(B, V, D):
    assert B % (NW * CHUNK) == 0
    b_per_w = B // NW
    n_chunks = b_per_w // CHUNK
    assert n_chunks % 2 == 0

    mesh = plsc.VectorSubcoreMesh(
        core_axis_name="c", subcore_axis_name="s", num_cores=NC, num_subcores=NS
    )

    @functools.partial(
        pl.kernel,
        mesh=mesh,
        out_type=jax.ShapeDtypeStruct((B, D), jnp.float32),
        scratch_types=[
            pltpu.VMEM((b_per_w,), jnp.int32),
            pltpu.VMEM((CHUNK, D), jnp.float32),
            pltpu.VMEM((CHUNK, D), jnp.float32),
            pltpu.SemaphoreType.DMA,
            pltpu.SemaphoreType.DMA,
        ],
    )
    def lookup(ids_hbm, table_hbm, out_hbm, idx_v, rows0, rows1, sem0, sem1):
        wid = lax.axis_index("s") * NC + lax.axis_index("c")
        base = wid * b_per_w
        pltpu.sync_copy(ids_hbm.at[pl.ds(base, b_per_w)], idx_v)

        def start(j, rows, sem):
            pltpu.async_copy(
                table_hbm.at[idx_v.at[pl.ds(j * CHUNK, CHUNK)]], rows, sem
            )

        def finish(j, rows, sem):
            pltpu.make_async_copy(
                table_hbm.at[idx_v.at[pl.ds(j * CHUNK, CHUNK)]], rows, sem
            ).wait()
            pltpu.sync_copy(rows, out_hbm.at[pl.ds(base + j * CHUNK, CHUNK)])

        start(0, rows0, sem0)

        def body(i, carry):
            g = i * 2
            start(g + 1, rows1, sem1)
            finish(g, rows0, sem0)

            @pl.when(g + 2 < n_chunks)
            def _():
                start(g + 2, rows0, sem0)

            finish(g + 1, rows1, sem1)
            return carry

        lax.fori_loop(0, n_chunks // 2, body, 0)

    return lookup


def kernel(enum_ids, table):
    B0, T = enum_ids.shape
    V, D = table.shape
    B = B0 * T
    ids = enum_ids.reshape(B).astype(jnp.int32)
    out = _make_lookup(B, V, D)(ids, table)
    return out.reshape(B0, T, D)


# speedup vs baseline: 1.0788x; 1.0788x over previous
import functools

import jax
import jax.numpy as jnp
from jax import lax
from jax.experimental import pallas as pl
from jax.experimental.pallas import tpu as pltpu
from jax.experimental.pallas import tpu_sc as plsc

NC = 2
NS = 16
NW = NC * NS
CHUNK = 128


def _make_lookup(B, V, D):
    assert B % (NW * CHUNK) == 0
    b_per_w = B // NW
    n_chunks = b_per_w // CHUNK
    assert n_chunks % 2 == 0

    mesh = plsc.VectorSubcoreMesh(
        core_axis_name="c", subcore_axis_name="s", num_cores=NC, num_subcores=NS
    )

    @functools.partial(
        pl.kernel,
        mesh=mesh,
        out_type=jax.ShapeDtypeStruct((B, D), jnp.float32),
        scratch_types=[
            pltpu.VMEM((b_per_w,), jnp.int32),
            pltpu.VMEM((CHUNK, D), jnp.float32),
            pltpu.VMEM((CHUNK, D), jnp.float32),
            pltpu.SemaphoreType.DMA,
            pltpu.SemaphoreType.DMA,
        ],
        compiler_params=pltpu.CompilerParams(use_tc_tiling_on_sc=False),
    )
    def lookup(ids_hbm, table_hbm, out_hbm, idx_v, rows0, rows1, sem0, sem1):
        wid = lax.axis_index("s") * NC + lax.axis_index("c")
        base = wid * b_per_w
        pltpu.sync_copy(ids_hbm.at[pl.ds(base, b_per_w)], idx_v)

        def start(j, rows, sem):
            pltpu.async_copy(
                table_hbm.at[idx_v.at[pl.ds(j * CHUNK, CHUNK)]], rows, sem
            )

        def finish(j, rows, sem):
            pltpu.make_async_copy(
                table_hbm.at[idx_v.at[pl.ds(j * CHUNK, CHUNK)]], rows, sem
            ).wait()
            pltpu.sync_copy(rows, out_hbm.at[pl.ds(base + j * CHUNK, CHUNK)])

        start(0, rows0, sem0)

        def body(i, carry):
            g = i * 2
            start(g + 1, rows1, sem1)
            finish(g, rows0, sem0)

            @pl.when(g + 2 < n_chunks)
            def _():
                start(g + 2, rows0, sem0)

            finish(g + 1, rows1, sem1)
            return carry

        lax.fori_loop(0, n_chunks // 2, body, 0)

    return lookup


def kernel(enum_ids, table):
    B0, T = enum_ids.shape
    V, D = table.shape
    B = B0 * T
    ids = enum_ids.reshape(B).astype(jnp.int32)
    out = _make_lookup(B, V, D)(ids, table)
    return out.reshape(B0, T, D)


# speedup vs baseline: 1.1105x; 1.0294x over previous
import functools

import jax
import jax.numpy as jnp
from jax import lax
from jax.experimental import pallas as pl
from jax.experimental.pallas import tpu as pltpu
from jax.experimental.pallas import tpu_sc as plsc

NC = 2
NS = 16
NW = NC * NS
CHUNK = 128


def _make_lookup(B, V, D):
    assert B % (NW * CHUNK) == 0
    b_per_w = B // NW
    n_chunks = b_per_w // CHUNK
    assert n_chunks % 2 == 0

    mesh = plsc.VectorSubcoreMesh(
        core_axis_name="c", subcore_axis_name="s", num_cores=NC, num_subcores=NS
    )

    NBUF = 8
    INFLIGHT = 4
    assert n_chunks % NBUF == 0

    @functools.partial(
        pl.kernel,
        mesh=mesh,
        out_type=jax.ShapeDtypeStruct((B, D), jnp.float32),
        scratch_types=[
            pltpu.VMEM((b_per_w,), jnp.int32),
            [pltpu.VMEM((CHUNK, D), jnp.float32) for _ in range(NBUF)],
            [pltpu.SemaphoreType.DMA for _ in range(NBUF)],
        ],
        compiler_params=pltpu.CompilerParams(use_tc_tiling_on_sc=False),
    )
    def lookup(ids_hbm, table_hbm, out_hbm, idx_v, rows, sems):
        wid = lax.axis_index("s") * NC + lax.axis_index("c")
        base = wid * b_per_w
        pltpu.sync_copy(ids_hbm.at[pl.ds(base, b_per_w)], idx_v)

        def gather(j, b):
            pltpu.async_copy(
                table_hbm.at[idx_v.at[pl.ds(j * CHUNK, CHUNK)]], rows[b], sems[b]
            )

        def wait_gather(j, b):
            pltpu.make_async_copy(
                table_hbm.at[idx_v.at[pl.ds(j * CHUNK, CHUNK)]], rows[b], sems[b]
            ).wait()

        def wb(j, b):
            pltpu.async_copy(rows[b], out_hbm.at[pl.ds(base + j * CHUNK, CHUNK)], sems[b])

        def wait_wb(j, b):
            pltpu.make_async_copy(
                rows[b], out_hbm.at[pl.ds(base + j * CHUNK, CHUNK)], sems[b]
            ).wait()

        for j0 in range(INFLIGHT):
            gather(j0, j0)

        def body(i, carry):
            for b in range(NBUF):
                j = i * NBUF + b
                wait_gather(j, b)
                wb(j, b)
                nb = (b + INFLIGHT) % NBUF

                @pl.when(j + INFLIGHT >= NBUF)
                def _():
                    wait_wb(j + INFLIGHT - NBUF, nb)

                @pl.when(j + INFLIGHT < n_chunks)
                def _():
                    gather(j + INFLIGHT, nb)

            return carry

        lax.fori_loop(0, n_chunks // NBUF, body, 0)

        for b in range(INFLIGHT, NBUF):
            wait_wb(n_chunks - NBUF + b, b)

    return lookup


def kernel(enum_ids, table):
    B0, T = enum_ids.shape
    V, D = table.shape
    B = B0 * T
    ids = enum_ids.reshape(B).astype(jnp.int32)
    out = _make_lookup(B, V, D)(ids, table)
    return out.reshape(B0, T, D)


# speedup vs baseline: 1.5059x; 1.3561x over previous
import functools

import jax
import jax.numpy as jnp
from jax import lax
from jax.experimental import pallas as pl
from jax.experimental.pallas import tpu as pltpu
from jax.experimental.pallas import tpu_sc as plsc

NC = 2
NS = 16
NW = NC * NS
CHUNK = 128


def _make_lookup(T, B0, V, D):
    B = T * B0
    DG, DR, BL = D // 8, 8, 128
    NBC = B0 // BL
    b_per_w = B // NW
    n_chunks = b_per_w // CHUNK
    NBUF = 4
    OBUF = 2

    mesh = plsc.VectorSubcoreMesh(
        core_axis_name="c", subcore_axis_name="s", num_cores=NC, num_subcores=NS
    )

    @functools.partial(
        pl.kernel,
        mesh=mesh,
        out_type=jax.ShapeDtypeStruct((T, DG, NBC, DR, BL), jnp.float32),
        scratch_types=[
            pltpu.VMEM((b_per_w,), jnp.int32),
            [pltpu.VMEM((CHUNK, D), jnp.float32) for _ in range(NBUF)],
            [pltpu.VMEM((1, DG, 1, DR, BL), jnp.float32) for _ in range(OBUF)],
            [pltpu.SemaphoreType.DMA for _ in range(NBUF)],
            [pltpu.SemaphoreType.DMA for _ in range(OBUF)],
        ],
        compiler_params=pltpu.CompilerParams(
            use_tc_tiling_on_sc=False, needs_layout_passes=False
        ),
    )
    def lookup(ids_hbm, table_hbm, out_hbm, idx_v, rows, obufs, gsems, wsems):
        wid = lax.axis_index("s") * NC + lax.axis_index("c")
        base = wid * b_per_w
        c0 = wid * n_chunks
        pltpu.sync_copy(ids_hbm.at[pl.ds(base, b_per_w)], idx_v)

        def gather(j, b):
            pltpu.async_copy(
                table_hbm.at[idx_v.at[pl.ds(j * CHUNK, CHUNK)]], rows[b], gsems[b]
            )

        def wait_gather(j, b):
            pltpu.make_async_copy(
                table_hbm.at[idx_v.at[pl.ds(j * CHUNK, CHUNK)]], rows[b], gsems[b]
            ).wait()

        def out_slice(j):
            c = c0 + j
            t = c // NBC
            bc = lax.rem(c, NBC)
            return out_hbm.at[pl.ds(t, 1), :, pl.ds(bc, 1)]

        def wb(j, ob):
            pltpu.async_copy(obufs[ob], out_slice(j), wsems[ob])

        def wait_wb(j, ob):
            pltpu.make_async_copy(obufs[ob], out_slice(j), wsems[ob]).wait()

        def transpose(b, ob):
            src = rows[b]
            dst = obufs[ob]
            for dg in range(DG):
                for dr in range(DR):
                    d = dg * DR + dr
                    dvec = jnp.full((16,), d, dtype=jnp.int32)
                    for g in range(BL // 16):
                        ridx = lax.iota(jnp.int32, 16) + (g * 16)
                        dst[0, dg, 0, dr, pl.ds(g * 16, 16)] = plsc.load_gather(
                            src, [ridx, dvec]
                        )

        for j0 in range(3):
            gather(j0, j0)

        def body(i, carry):
            for u in range(4):
                j = i * 4 + u
                b = u
                ob = u % OBUF

                @pl.when(j >= OBUF)
                def _():
                    wait_wb(j - OBUF, ob)

                wait_gather(j, b)
                transpose(b, ob)
                wb(j, ob)

                @pl.when(j + 3 < n_chunks)
                def _():
                    gather(j + 3, (u + 3) % NBUF)

            return carry

        lax.fori_loop(0, n_chunks // 4, body, 0)

        wait_wb(n_chunks - 2, (n_chunks - 2) % OBUF)
        wait_wb(n_chunks - 1, (n_chunks - 1) % OBUF)

    return lookup


def kernel(enum_ids, table):
    B0, T = enum_ids.shape
    V, D = table.shape
    ids = enum_ids.T.reshape(T * B0).astype(jnp.int32)
    out5 = _make_lookup(T, B0, V, D)(ids, table)
    return out5.transpose(2, 4, 0, 1, 3).reshape(B0, T, D)
